# baseline (speedup 1.0000x reference)
import functools

import jax
import jax.numpy as jnp
from jax import lax
from jax.experimental import pallas as pl
from jax.experimental.pallas import tpu as pltpu
from jax.experimental.pallas import tpu_sc as plsc

B = 4096
L = 200
LP = 208
HALF = LP // 2
D = 64
C = 2
V = 1000000
NC = 2
NS = 16
NW = NC * NS
BPW = B // NW
LANES = 16
NBUF = 8
GB = 512
GP = -(-(-(-V // 128)) // GB) * GB


def _tc_project(tableT, w8):
    CB = GB * 128
    NBLK = GP * 128 // CB

    def body(tbl_ref, w_ref, out_ref):
        w2 = w_ref[pl.ds(0, 2), :].astype(jnp.bfloat16)
        blk = tbl_ref[...].astype(jnp.bfloat16)
        p = lax.dot_general(w2, blk, (((1,), (0,)), ((), ())),
                            preferred_element_type=jnp.float32)
        pb = lax.bitcast_convert_type(
            p.astype(jnp.bfloat16), jnp.uint16).astype(jnp.uint32)
        word = (pb[0:1, :] << 16) | pb[1:2, :]
        out_ref[0, :, :] = lax.bitcast_convert_type(word, jnp.float32)

    return pl.pallas_call(
        body,
        grid=(NBLK,),
        in_specs=[
            pl.BlockSpec((D, CB), lambda i: (0, i)),
            pl.BlockSpec((8, D), lambda i: (0, 0)),
        ],
        out_specs=pl.BlockSpec((1, 1, CB), lambda i: (i, 0, 0)),
        out_shape=jax.ShapeDtypeStruct((NBLK, 1, CB), jnp.float32),
    )(tableT, w8)


def _sc_pair_sums(ids_p, p1d):
    mesh = plsc.VectorSubcoreMesh(
        core_axis_name="c", subcore_axis_name="s",
        num_cores=NC, num_subcores=NS)

    @functools.partial(
        pl.kernel,
        out_type=jax.ShapeDtypeStruct((B, 2 * LANES), jnp.float32),
        mesh=mesh,
        scratch_types=(
            [pltpu.VMEM((BPW, LP), jnp.int32)]
            + [pltpu.VMEM((LP,), jnp.float32)] * NBUF
            + [pltpu.VMEM((BPW, 2 * LANES), jnp.float32)]
            + [pltpu.SemaphoreType.DMA] * NBUF
        ),
        compiler_params=pltpu.CompilerParams(
            use_tc_tiling_on_sc=False, needs_layout_passes=False),
    )
    def k(ids_hbm, p_hbm, out_hbm, ids_v, *rest):
        bufs = rest[:NBUF]
        sums_v = rest[NBUF]
        sems = rest[NBUF + 1:]
        wid = lax.axis_index("s") * NC + lax.axis_index("c")
        base = wid * BPW
        pltpu.sync_copy(ids_hbm.at[pl.ds(base, BPW), :], ids_v)

        def copies(r, sbuf, sem):
            return (
                pltpu.make_async_copy(
                    p_hbm.at[ids_v.at[r, pl.ds(0, HALF)]],
                    sbuf.at[pl.ds(0, HALF)], sem),
                pltpu.make_async_copy(
                    p_hbm.at[ids_v.at[r, pl.ds(HALF, HALF)]],
                    sbuf.at[pl.ds(HALF, HALF)], sem),
            )

        def issue(r, j):
            c0, c1 = copies(r, bufs[j], sems[j])
            c0.start()
            c1.start()

        def wait(r, j):
            c0, c1 = copies(r, bufs[j], sems[j])
            c0.wait()
            c1.wait()

        tailmask = lax.iota(jnp.int32, LANES) < (L % LANES)

        def accum(r, sbuf):
            z = jnp.zeros((LANES,), jnp.float32)

            def unpacked(t):
                words = sbuf[pl.ds(LANES * t, LANES)]
                pairs = plsc.bitcast(words, jnp.bfloat16)
                return plsc.unpack(pairs, format=plsc.PackFormat.INTERLEAVED)

            def body(t, a):
                a1, a0 = a
                lo, hi = unpacked(t)
                return (a1 + lo, a0 + hi)

            acc1, acc0 = lax.fori_loop(0, L // LANES, body, (z, z), unroll=4)
            lo, hi = unpacked(L // LANES)
            acc1 = acc1 + jnp.where(tailmask, lo, 0.0)
            acc0 = acc0 + jnp.where(tailmask, hi, 0.0)
            sums_v[r, pl.ds(0, LANES)] = acc0
            sums_v[r, pl.ds(LANES, LANES)] = acc1

        for j in range(NBUF - 1):
            issue(j, j)

        def step(i, carry):
            rr = i * NBUF
            for j in range(NBUF):
                r = rr + j
                jn = (j + NBUF - 1) % NBUF

                @pl.when(r + NBUF - 1 < BPW)
                def _():
                    issue(r + NBUF - 1, jn)

                wait(r, j)
                accum(r, bufs[j])
            return carry

        lax.fori_loop(0, BPW // NBUF, step, 0)
        pltpu.sync_copy(sums_v, out_hbm.at[pl.ds(base, BPW), :])

    return k(ids_p, p1d)


def _tc_head(ids, pair_sums, sel, b8):
    R = 512

    def body(ids_ref, ps_ref, sel_ref, b_ref, out_ref):
        cnt = jnp.sum((ids_ref[...] != 0).astype(jnp.float32),
                      axis=1, keepdims=True)
        tot = lax.dot_general(ps_ref[...], sel_ref[...],
                              (((1,), (0,)), ((), ())),
                              preferred_element_type=jnp.float32)
        out_ref[...] = tot / (cnt + 1e-8) + b_ref[...]

    return pl.pallas_call(
        body,
        grid=(B // R,),
        in_specs=[
            pl.BlockSpec((R, L), lambda i: (i, 0)),
            pl.BlockSpec((R, 2 * LANES), lambda i: (i, 0)),
            pl.BlockSpec((2 * LANES, 8), lambda i: (0, 0)),
            pl.BlockSpec((1, 8), lambda i: (0, 0)),
        ],
        out_specs=pl.BlockSpec((R, 8), lambda i: (i, 0)),
        out_shape=jax.ShapeDtypeStruct((B, 8), jnp.float32),
    )(ids, pair_sums, sel, b8)


def kernel(input_ids, table, W, b):
    ids = input_ids.astype(jnp.int32)
    npad = LP - L
    pad_ids = (jnp.arange(B * npad, dtype=jnp.int32).reshape(B, npad)
               * 997) % table.shape[0]
    ids_p = jnp.concatenate([ids, pad_ids], axis=1)

    w8 = jnp.pad(W.astype(jnp.float32), ((0, 8 - C), (0, 0)))
    p2 = _tc_project(table.astype(jnp.float32).T, w8)
    p1d = p2.reshape(GP * 128)

    pair_sums = _sc_pair_sums(ids_p, p1d)

    lane = jnp.arange(2 * LANES)
    cls = jnp.arange(8)
    sel = ((lane[:, None] // LANES) == cls[None, :]).astype(jnp.float32)
    b8 = jnp.pad(b.astype(jnp.float32), (0, 8 - C)).reshape(1, 8)
    logits8 = _tc_head(ids, pair_sums, sel, b8)
    return logits8[:, :C]


# speedup vs baseline: 1.0168x; 1.0168x over previous
import functools

import jax
import jax.numpy as jnp
from jax import lax
from jax.experimental import pallas as pl
from jax.experimental.pallas import tpu as pltpu
from jax.experimental.pallas import tpu_sc as plsc

B = 4096
L = 200
LP = 208
HALF = LP // 2
D = 64
C = 2
V = 1000000
NC = 2
NS = 16
NW = NC * NS
BPW = B // NW
LANES = 16
NBUF = 8
GB = 256
GP = -(-(-(-V // 128)) // GB) * GB


def _tc_project(tableT, w8):
    CB = GB * 128
    NBLK = GP * 128 // CB

    def body(tbl_ref, w_ref, out_ref):
        w2 = w_ref[pl.ds(0, 2), :].astype(jnp.bfloat16)
        blk = tbl_ref[...].astype(jnp.bfloat16)
        p = lax.dot_general(w2, blk, (((1,), (0,)), ((), ())),
                            preferred_element_type=jnp.float32)
        pb = lax.bitcast_convert_type(
            p.astype(jnp.bfloat16), jnp.uint16).astype(jnp.uint32)
        word = (pb[0:1, :] << 16) | pb[1:2, :]
        out_ref[0, :, :] = lax.bitcast_convert_type(word, jnp.float32)

    return pl.pallas_call(
        body,
        grid=(NBLK,),
        in_specs=[
            pl.BlockSpec((D, CB), lambda i: (0, i)),
            pl.BlockSpec((8, D), lambda i: (0, 0)),
        ],
        out_specs=pl.BlockSpec((1, 1, CB), lambda i: (i, 0, 0)),
        out_shape=jax.ShapeDtypeStruct((NBLK, 1, CB), jnp.float32),
    )(tableT, w8)


def _sc_pair_sums(ids_p, p1d):
    mesh = plsc.VectorSubcoreMesh(
        core_axis_name="c", subcore_axis_name="s",
        num_cores=NC, num_subcores=NS)

    @functools.partial(
        pl.kernel,
        out_type=jax.ShapeDtypeStruct((B, 2 * LANES), jnp.float32),
        mesh=mesh,
        scratch_types=(
            [pltpu.VMEM((BPW, LP), jnp.int32)]
            + [pltpu.VMEM((LP,), jnp.float32)] * NBUF
            + [pltpu.VMEM((BPW, 2 * LANES), jnp.float32)]
            + [pltpu.SemaphoreType.DMA] * NBUF
        ),
        compiler_params=pltpu.CompilerParams(
            use_tc_tiling_on_sc=False, needs_layout_passes=False),
    )
    def k(ids_hbm, p_hbm, out_hbm, ids_v, *rest):
        bufs = rest[:NBUF]
        sums_v = rest[NBUF]
        sems = rest[NBUF + 1:]
        wid = lax.axis_index("s") * NC + lax.axis_index("c")
        base = wid * BPW
        pltpu.sync_copy(ids_hbm.at[pl.ds(base, BPW), :], ids_v)

        def copies(r, sbuf, sem):
            return (
                pltpu.make_async_copy(
                    p_hbm.at[ids_v.at[r, pl.ds(0, HALF)]],
                    sbuf.at[pl.ds(0, HALF)], sem),
                pltpu.make_async_copy(
                    p_hbm.at[ids_v.at[r, pl.ds(HALF, HALF)]],
                    sbuf.at[pl.ds(HALF, HALF)], sem),
            )

        def issue(r, j):
            c0, c1 = copies(r, bufs[j], sems[j])
            c0.start()
            c1.start()

        def wait(r, j):
            c0, c1 = copies(r, bufs[j], sems[j])
            c0.wait()
            c1.wait()

        tailmask = lax.iota(jnp.int32, LANES) < (L % LANES)

        def accum(r, sbuf):
            z = jnp.zeros((LANES,), jnp.float32)

            def unpacked(t):
                words = sbuf[pl.ds(LANES * t, LANES)]
                pairs = plsc.bitcast(words, jnp.bfloat16)
                return plsc.unpack(pairs, format=plsc.PackFormat.INTERLEAVED)

            def body(t, a):
                a1, a0 = a
                lo, hi = unpacked(t)
                return (a1 + lo, a0 + hi)

            acc1, acc0 = lax.fori_loop(0, L // LANES, body, (z, z), unroll=4)
            lo, hi = unpacked(L // LANES)
            acc1 = acc1 + jnp.where(tailmask, lo, 0.0)
            acc0 = acc0 + jnp.where(tailmask, hi, 0.0)
            sums_v[r, pl.ds(0, LANES)] = acc0
            sums_v[r, pl.ds(LANES, LANES)] = acc1

        for j in range(NBUF - 1):
            issue(j, j)

        def step(i, carry):
            rr = i * NBUF
            for j in range(NBUF):
                r = rr + j
                jn = (j + NBUF - 1) % NBUF

                @pl.when(r + NBUF - 1 < BPW)
                def _():
                    issue(r + NBUF - 1, jn)

                wait(r, j)
                accum(r, bufs[j])
            return carry

        lax.fori_loop(0, BPW // NBUF, step, 0)
        pltpu.sync_copy(sums_v, out_hbm.at[pl.ds(base, BPW), :])

    return k(ids_p, p1d)


def _tc_head(ids, pair_sums, sel, b8):
    R = 512

    def body(ids_ref, ps_ref, sel_ref, b_ref, out_ref):
        cnt = jnp.sum((ids_ref[...] != 0).astype(jnp.float32),
                      axis=1, keepdims=True)
        tot = lax.dot_general(ps_ref[...], sel_ref[...],
                              (((1,), (0,)), ((), ())),
                              preferred_element_type=jnp.float32)
        out_ref[...] = tot / (cnt + 1e-8) + b_ref[...]

    return pl.pallas_call(
        body,
        grid=(B // R,),
        in_specs=[
            pl.BlockSpec((R, L), lambda i: (i, 0)),
            pl.BlockSpec((R, 2 * LANES), lambda i: (i, 0)),
            pl.BlockSpec((2 * LANES, 8), lambda i: (0, 0)),
            pl.BlockSpec((1, 8), lambda i: (0, 0)),
        ],
        out_specs=pl.BlockSpec((R, 8), lambda i: (i, 0)),
        out_shape=jax.ShapeDtypeStruct((B, 8), jnp.float32),
    )(ids, pair_sums, sel, b8)


def kernel(input_ids, table, W, b):
    ids = input_ids.astype(jnp.int32)
    npad = LP - L
    pad_ids = (jnp.arange(B * npad, dtype=jnp.int32).reshape(B, npad)
               * 997) % table.shape[0]
    ids_p = jnp.concatenate([ids, pad_ids], axis=1)

    w8 = jnp.pad(W.astype(jnp.float32), ((0, 8 - C), (0, 0)))
    p2 = _tc_project(table.astype(jnp.float32).T, w8)
    p1d = p2.reshape(GP * 128)

    pair_sums = _sc_pair_sums(ids_p, p1d)

    lane = jnp.arange(2 * LANES)
    cls = jnp.arange(8)
    sel = ((lane[:, None] // LANES) == cls[None, :]).astype(jnp.float32)
    b8 = jnp.pad(b.astype(jnp.float32), (0, 8 - C)).reshape(1, 8)
    logits8 = _tc_head(ids, pair_sums, sel, b8)
    return logits8[:, :C]


# speedup vs baseline: 1.0250x; 1.0081x over previous
import functools

import jax
import jax.numpy as jnp
from jax import lax
from jax.experimental import pallas as pl
from jax.experimental.pallas import tpu as pltpu
from jax.experimental.pallas import tpu_sc as plsc

B = 4096
L = 200
LP = 208
HALF = LP // 2
D = 64
C = 2
V = 1000000
NC = 2
NS = 16
NW = NC * NS
BPW = B // NW
LANES = 16
NBUF = 16
GB = 256
GP = -(-(-(-V // 128)) // GB) * GB


def _tc_project(tableT, w8):
    CB = GB * 128
    NBLK = GP * 128 // CB

    def body(tbl_ref, w_ref, out_ref):
        w2 = w_ref[pl.ds(0, 2), :].astype(jnp.bfloat16)
        blk = tbl_ref[...].astype(jnp.bfloat16)
        p = lax.dot_general(w2, blk, (((1,), (0,)), ((), ())),
                            preferred_element_type=jnp.float32)
        pb = lax.bitcast_convert_type(
            p.astype(jnp.bfloat16), jnp.uint16).astype(jnp.uint32)
        word = (pb[0:1, :] << 16) | pb[1:2, :]
        out_ref[0, :, :] = lax.bitcast_convert_type(word, jnp.float32)

    return pl.pallas_call(
        body,
        grid=(NBLK,),
        in_specs=[
            pl.BlockSpec((D, CB), lambda i: (0, i)),
            pl.BlockSpec((8, D), lambda i: (0, 0)),
        ],
        out_specs=pl.BlockSpec((1, 1, CB), lambda i: (i, 0, 0)),
        out_shape=jax.ShapeDtypeStruct((NBLK, 1, CB), jnp.float32),
    )(tableT, w8)


def _sc_pair_sums(ids_p, p1d):
    mesh = plsc.VectorSubcoreMesh(
        core_axis_name="c", subcore_axis_name="s",
        num_cores=NC, num_subcores=NS)

    @functools.partial(
        pl.kernel,
        out_type=jax.ShapeDtypeStruct((B, 2 * LANES), jnp.float32),
        mesh=mesh,
        scratch_types=(
            [pltpu.VMEM((BPW, LP), jnp.int32)]
            + [pltpu.VMEM((LP,), jnp.float32)] * NBUF
            + [pltpu.VMEM((BPW, 2 * LANES), jnp.float32)]
            + [pltpu.SemaphoreType.DMA] * NBUF
        ),
        compiler_params=pltpu.CompilerParams(
            use_tc_tiling_on_sc=False, needs_layout_passes=False),
    )
    def k(ids_hbm, p_hbm, out_hbm, ids_v, *rest):
        bufs = rest[:NBUF]
        sums_v = rest[NBUF]
        sems = rest[NBUF + 1:]
        wid = lax.axis_index("s") * NC + lax.axis_index("c")
        base = wid * BPW
        pltpu.sync_copy(ids_hbm.at[pl.ds(base, BPW), :], ids_v)

        def copies(r, sbuf, sem):
            return (
                pltpu.make_async_copy(
                    p_hbm.at[ids_v.at[r, pl.ds(0, HALF)]],
                    sbuf.at[pl.ds(0, HALF)], sem),
                pltpu.make_async_copy(
                    p_hbm.at[ids_v.at[r, pl.ds(HALF, HALF)]],
                    sbuf.at[pl.ds(HALF, HALF)], sem),
            )

        def issue(r, j):
            c0, c1 = copies(r, bufs[j], sems[j])
            c0.start()
            c1.start()

        def wait(r, j):
            c0, c1 = copies(r, bufs[j], sems[j])
            c0.wait()
            c1.wait()

        tailmask = lax.iota(jnp.int32, LANES) < (L % LANES)

        def accum(r, sbuf):
            z = jnp.zeros((LANES,), jnp.float32)

            def unpacked(t):
                words = sbuf[pl.ds(LANES * t, LANES)]
                pairs = plsc.bitcast(words, jnp.bfloat16)
                return plsc.unpack(pairs, format=plsc.PackFormat.INTERLEAVED)

            def body(t, a):
                a1, a0 = a
                lo, hi = unpacked(t)
                return (a1 + lo, a0 + hi)

            acc1, acc0 = lax.fori_loop(0, L // LANES, body, (z, z), unroll=4)
            lo, hi = unpacked(L // LANES)
            acc1 = acc1 + jnp.where(tailmask, lo, 0.0)
            acc0 = acc0 + jnp.where(tailmask, hi, 0.0)
            sums_v[r, pl.ds(0, LANES)] = acc0
            sums_v[r, pl.ds(LANES, LANES)] = acc1

        for j in range(NBUF - 1):
            issue(j, j)

        def step(i, carry):
            rr = i * NBUF
            for j in range(NBUF):
                r = rr + j
                jn = (j + NBUF - 1) % NBUF

                @pl.when(r + NBUF - 1 < BPW)
                def _():
                    issue(r + NBUF - 1, jn)

                wait(r, j)
                accum(r, bufs[j])
            return carry

        lax.fori_loop(0, BPW // NBUF, step, 0)
        pltpu.sync_copy(sums_v, out_hbm.at[pl.ds(base, BPW), :])

    return k(ids_p, p1d)


def _tc_head(ids, pair_sums, sel, b8):
    R = 512

    def body(ids_ref, ps_ref, sel_ref, b_ref, out_ref):
        cnt = jnp.sum((ids_ref[...] != 0).astype(jnp.float32),
                      axis=1, keepdims=True) - float(LP - L)
        tot = lax.dot_general(ps_ref[...], sel_ref[...],
                              (((1,), (0,)), ((), ())),
                              preferred_element_type=jnp.float32)
        out_ref[...] = tot / (cnt + 1e-8) + b_ref[...]

    return pl.pallas_call(
        body,
        grid=(B // R,),
        in_specs=[
            pl.BlockSpec((R, LP), lambda i: (i, 0)),
            pl.BlockSpec((R, 2 * LANES), lambda i: (i, 0)),
            pl.BlockSpec((2 * LANES, 8), lambda i: (0, 0)),
            pl.BlockSpec((1, 8), lambda i: (0, 0)),
        ],
        out_specs=pl.BlockSpec((R, 8), lambda i: (i, 0)),
        out_shape=jax.ShapeDtypeStruct((B, 8), jnp.float32),
    )(ids, pair_sums, sel, b8)


def kernel(input_ids, table, W, b):
    ids = input_ids.astype(jnp.int32)
    npad = LP - L
    pad_ids = (jnp.arange(B * npad, dtype=jnp.int32).reshape(B, npad)
               * 997) % (table.shape[0] - 1) + 1
    ids_p = jnp.concatenate([ids, pad_ids], axis=1)

    w8 = jnp.pad(W.astype(jnp.float32), ((0, 8 - C), (0, 0)))
    p2 = _tc_project(table.astype(jnp.float32).T, w8)
    p1d = p2.reshape(GP * 128)

    pair_sums = _sc_pair_sums(ids_p, p1d)

    lane = jnp.arange(2 * LANES)
    cls = jnp.arange(8)
    sel = ((lane[:, None] // LANES) == cls[None, :]).astype(jnp.float32)
    b8 = jnp.pad(b.astype(jnp.float32), (0, 8 - C)).reshape(1, 8)
    logits8 = _tc_head(ids_p, pair_sums, sel, b8)
    return logits8[:, :C]


# speedup vs baseline: 1.0677x; 1.0417x over previous
import functools

import jax
import jax.numpy as jnp
from jax import lax
from jax.experimental import pallas as pl
from jax.experimental.pallas import tpu as pltpu
from jax.experimental.pallas import tpu_sc as plsc

B = 4096
L = 200
LP = 208
HALF = LP // 2
D = 64
C = 2
V = 1000000
NC = 2
NS = 16
NW = NC * NS
BPW = B // NW
LANES = 16
NBUF = 16
GB = 256
GP = -(-(-(-V // 128)) // GB) * GB


def _tc_project(tableT, w8):
    CB = GB * 128
    NBLK = GP * 128 // CB

    def body(tbl_ref, w_ref, out_ref):
        w2 = w_ref[pl.ds(0, 2), :].astype(jnp.bfloat16)
        blk = tbl_ref[...].astype(jnp.bfloat16)
        p = lax.dot_general(w2, blk, (((1,), (0,)), ((), ())),
                            preferred_element_type=jnp.float32)
        pb = lax.bitcast_convert_type(
            p.astype(jnp.bfloat16), jnp.uint16).astype(jnp.uint32)
        word = (pb[0:1, :] << 16) | pb[1:2, :]
        out_ref[0, :, :] = lax.bitcast_convert_type(word, jnp.float32)

    return pl.pallas_call(
        body,
        grid=(NBLK,),
        in_specs=[
            pl.BlockSpec((D, CB), lambda i: (0, i)),
            pl.BlockSpec((8, D), lambda i: (0, 0)),
        ],
        out_specs=pl.BlockSpec((1, 1, CB), lambda i: (i, 0, 0)),
        out_shape=jax.ShapeDtypeStruct((NBLK, 1, CB), jnp.float32),
    )(tableT, w8)


def _sc_logits(ids_p, p1d, b8):
    mesh = plsc.VectorSubcoreMesh(
        core_axis_name="c", subcore_axis_name="s",
        num_cores=NC, num_subcores=NS)

    @functools.partial(
        pl.kernel,
        out_type=jax.ShapeDtypeStruct((B, LANES), jnp.float32),
        mesh=mesh,
        scratch_types=(
            [pltpu.VMEM((BPW, LP), jnp.int32)]
            + [pltpu.VMEM((LP,), jnp.float32)] * NBUF
            + [pltpu.VMEM((BPW, LANES), jnp.float32)]
            + [pltpu.VMEM((LANES,), jnp.float32)]
            + [pltpu.SemaphoreType.DMA] * NBUF
        ),
        compiler_params=pltpu.CompilerParams(
            use_tc_tiling_on_sc=False, needs_layout_passes=False),
    )
    def k(ids_hbm, p_hbm, b_hbm, out_hbm, ids_v, *rest):
        bufs = rest[:NBUF]
        sums_v = rest[NBUF]
        b_v = rest[NBUF + 1]
        sems = rest[NBUF + 2:]
        wid = lax.axis_index("s") * NC + lax.axis_index("c")
        base = wid * BPW
        pltpu.sync_copy(b_hbm, b_v)
        pltpu.sync_copy(ids_hbm.at[pl.ds(base, BPW), :], ids_v)

        def copies(r, sbuf, sem):
            return (
                pltpu.make_async_copy(
                    p_hbm.at[ids_v.at[r, pl.ds(0, HALF)]],
                    sbuf.at[pl.ds(0, HALF)], sem),
                pltpu.make_async_copy(
                    p_hbm.at[ids_v.at[r, pl.ds(HALF, HALF)]],
                    sbuf.at[pl.ds(HALF, HALF)], sem),
            )

        def issue(r, j):
            c0, c1 = copies(r, bufs[j], sems[j])
            c0.start()
            c1.start()

        def wait(r, j):
            c0, c1 = copies(r, bufs[j], sems[j])
            c0.wait()
            c1.wait()

        tailmask = lax.iota(jnp.int32, LANES) < (L % LANES)
        lane0 = lax.iota(jnp.int32, LANES) == 0

        def accum(r, sbuf):
            z = jnp.zeros((LANES,), jnp.float32)

            def unpacked(t):
                words = sbuf[pl.ds(LANES * t, LANES)]
                pairs = plsc.bitcast(words, jnp.bfloat16)
                return plsc.unpack(pairs, format=plsc.PackFormat.INTERLEAVED)

            def nonpad(t):
                return (ids_v[r, pl.ds(LANES * t, LANES)] != 0).astype(
                    jnp.float32)

            def body(t, a):
                a1, a0, ac = a
                lo, hi = unpacked(t)
                return (a1 + lo, a0 + hi, ac + nonpad(t))

            acc1, acc0, cntv = lax.fori_loop(
                0, L // LANES, body, (z, z, z), unroll=4)
            lo, hi = unpacked(L // LANES)
            acc1 = acc1 + jnp.where(tailmask, lo, 0.0)
            acc0 = acc0 + jnp.where(tailmask, hi, 0.0)
            cntv = cntv + jnp.where(tailmask, nonpad(L // LANES), 0.0)
            s0v = jnp.full((LANES,), jnp.sum(acc0))
            s1v = jnp.full((LANES,), jnp.sum(acc1))
            cnv = jnp.full((LANES,), jnp.sum(cntv))
            res = jnp.where(lane0, s0v, s1v) / (cnv + 1e-8) + b_v[...]
            sums_v[r, pl.ds(0, LANES)] = res

        for j in range(NBUF - 1):
            issue(j, j)

        def step(i, carry):
            rr = i * NBUF
            for j in range(NBUF):
                r = rr + j
                jn = (j + NBUF - 1) % NBUF

                @pl.when(r + NBUF - 1 < BPW)
                def _():
                    issue(r + NBUF - 1, jn)

                wait(r, j)
                accum(r, bufs[j])
            return carry

        lax.fori_loop(0, BPW // NBUF, step, 0)
        pltpu.sync_copy(sums_v, out_hbm.at[pl.ds(base, BPW), :])

    return k(ids_p, p1d, b8)


def _tc_head(ids, pair_sums, sel, b8):
    R = 512

    def body(ids_ref, ps_ref, sel_ref, b_ref, out_ref):
        cnt = jnp.sum((ids_ref[...] != 0).astype(jnp.float32),
                      axis=1, keepdims=True) - float(LP - L)
        tot = lax.dot_general(ps_ref[...], sel_ref[...],
                              (((1,), (0,)), ((), ())),
                              preferred_element_type=jnp.float32)
        out_ref[...] = tot / (cnt + 1e-8) + b_ref[...]

    return pl.pallas_call(
        body,
        grid=(B // R,),
        in_specs=[
            pl.BlockSpec((R, LP), lambda i: (i, 0)),
            pl.BlockSpec((R, 2 * LANES), lambda i: (i, 0)),
            pl.BlockSpec((2 * LANES, 8), lambda i: (0, 0)),
            pl.BlockSpec((1, 8), lambda i: (0, 0)),
        ],
        out_specs=pl.BlockSpec((R, 8), lambda i: (i, 0)),
        out_shape=jax.ShapeDtypeStruct((B, 8), jnp.float32),
    )(ids, pair_sums, sel, b8)


def kernel(input_ids, table, W, b):
    ids = input_ids.astype(jnp.int32)
    npad = LP - L
    pad_ids = (jnp.arange(B * npad, dtype=jnp.int32).reshape(B, npad)
               * 997) % (table.shape[0] - 1) + 1
    ids_p = jnp.concatenate([ids, pad_ids], axis=1)

    w8 = jnp.pad(W.astype(jnp.float32), ((0, 8 - C), (0, 0)))
    p2 = _tc_project(table.astype(jnp.float32).T, w8)
    p1d = p2.reshape(GP * 128)

    b16 = jnp.where(jnp.arange(LANES) == 0, b[0], b[1]).astype(jnp.float32)
    return _sc_logits(ids_p, p1d, b16)[:, :C]


# speedup vs baseline: 1.0895x; 1.0204x over previous
import functools

import jax
import jax.numpy as jnp
from jax import lax
from jax.experimental import pallas as pl
from jax.experimental.pallas import tpu as pltpu
from jax.experimental.pallas import tpu_sc as plsc

B = 4096
L = 200
LP = 208
HALF = LP // 2
D = 64
C = 2
V = 1000000
NC = 2
NS = 16
NW = NC * NS
BPW = B // NW
LANES = 16
NBUF = 16
GB = 256
GP = -(-(-(-V // 128)) // GB) * GB


def _tc_project(tableT, w8):
    CB = GB * 128
    NBLK = GP * 128 // CB

    def body(tbl_ref, w_ref, out_ref):
        w2 = w_ref[pl.ds(0, 2), :].astype(jnp.bfloat16)
        blk = tbl_ref[...].astype(jnp.bfloat16)
        p = lax.dot_general(w2, blk, (((1,), (0,)), ((), ())),
                            preferred_element_type=jnp.float32)
        pb = lax.bitcast_convert_type(
            p.astype(jnp.bfloat16), jnp.uint16).astype(jnp.uint32)
        word = (pb[0:1, :] << 16) | pb[1:2, :]
        out_ref[0, :, :] = lax.bitcast_convert_type(word, jnp.float32)

    return pl.pallas_call(
        body,
        grid=(NBLK,),
        in_specs=[
            pl.BlockSpec((D, CB), lambda i: (0, i)),
            pl.BlockSpec((8, D), lambda i: (0, 0)),
        ],
        out_specs=pl.BlockSpec((1, 1, CB), lambda i: (i, 0, 0)),
        out_shape=jax.ShapeDtypeStruct((NBLK, 1, CB), jnp.float32),
    )(tableT, w8)


def _sc_logits(ids_p, p1d, b8):
    mesh = plsc.VectorSubcoreMesh(
        core_axis_name="c", subcore_axis_name="s",
        num_cores=NC, num_subcores=NS)

    @functools.partial(
        pl.kernel,
        out_type=jax.ShapeDtypeStruct((B, LANES), jnp.float32),
        mesh=mesh,
        scratch_types=(
            [pltpu.VMEM((BPW * LP,), jnp.int32)]
            + [pltpu.VMEM((LP,), jnp.float32)] * NBUF
            + [pltpu.VMEM((BPW, LANES), jnp.float32)]
            + [pltpu.VMEM((LANES,), jnp.float32)]
            + [pltpu.SemaphoreType.DMA] * NBUF
        ),
        compiler_params=pltpu.CompilerParams(
            use_tc_tiling_on_sc=False, needs_layout_passes=False),
    )
    def k(ids_hbm, p_hbm, b_hbm, out_hbm, ids_v, *rest):
        bufs = rest[:NBUF]
        sums_v = rest[NBUF]
        b_v = rest[NBUF + 1]
        sems = rest[NBUF + 2:]
        wid = lax.axis_index("s") * NC + lax.axis_index("c")
        base = wid * BPW
        pltpu.sync_copy(b_hbm, b_v)
        pltpu.sync_copy(ids_hbm.at[pl.ds(base * LP, BPW * LP)], ids_v)

        def copies(r, sbuf, sem):
            return (
                pltpu.make_async_copy(
                    p_hbm.at[ids_v.at[pl.ds(r * LP, HALF)]],
                    sbuf.at[pl.ds(0, HALF)], sem),
                pltpu.make_async_copy(
                    p_hbm.at[ids_v.at[pl.ds(r * LP + HALF, HALF)]],
                    sbuf.at[pl.ds(HALF, HALF)], sem),
            )

        def issue(r, j):
            c0, c1 = copies(r, bufs[j], sems[j])
            c0.start()
            c1.start()

        def wait(r, j):
            c0, c1 = copies(r, bufs[j], sems[j])
            c0.wait()
            c1.wait()

        tailmask = lax.iota(jnp.int32, LANES) < (L % LANES)
        lane0 = lax.iota(jnp.int32, LANES) == 0

        def accum(r, sbuf):
            z = jnp.zeros((LANES,), jnp.float32)

            def unpacked(t):
                words = sbuf[pl.ds(LANES * t, LANES)]
                pairs = plsc.bitcast(words, jnp.bfloat16)
                return plsc.unpack(pairs, format=plsc.PackFormat.INTERLEAVED)

            def nonpad(t):
                return (ids_v[pl.ds(r * LP + LANES * t, LANES)] != 0).astype(
                    jnp.float32)

            def body(t, a):
                a1, a0, ac = a
                lo, hi = unpacked(t)
                return (a1 + lo, a0 + hi, ac + nonpad(t))

            acc1, acc0, cntv = lax.fori_loop(
                0, L // LANES, body, (z, z, z), unroll=4)
            lo, hi = unpacked(L // LANES)
            acc1 = acc1 + jnp.where(tailmask, lo, 0.0)
            acc0 = acc0 + jnp.where(tailmask, hi, 0.0)
            cntv = cntv + jnp.where(tailmask, nonpad(L // LANES), 0.0)
            s0v = jnp.full((LANES,), jnp.sum(acc0))
            s1v = jnp.full((LANES,), jnp.sum(acc1))
            cnv = jnp.full((LANES,), jnp.sum(cntv))
            res = jnp.where(lane0, s0v, s1v) / (cnv + 1e-8) + b_v[...]
            sums_v[r, pl.ds(0, LANES)] = res

        for j in range(NBUF - 1):
            issue(j, j)

        def step(i, carry):
            rr = i * NBUF
            for j in range(NBUF):
                r = rr + j
                jn = (j + NBUF - 1) % NBUF

                @pl.when(r + NBUF - 1 < BPW)
                def _():
                    issue(r + NBUF - 1, jn)

                wait(r, j)
                accum(r, bufs[j])
            return carry

        lax.fori_loop(0, BPW // NBUF, step, 0)
        pltpu.sync_copy(sums_v, out_hbm.at[pl.ds(base, BPW), :])

    return k(ids_p, p1d, b8)


def _tc_head(ids, pair_sums, sel, b8):
    R = 512

    def body(ids_ref, ps_ref, sel_ref, b_ref, out_ref):
        cnt = jnp.sum((ids_ref[...] != 0).astype(jnp.float32),
                      axis=1, keepdims=True) - float(LP - L)
        tot = lax.dot_general(ps_ref[...], sel_ref[...],
                              (((1,), (0,)), ((), ())),
                              preferred_element_type=jnp.float32)
        out_ref[...] = tot / (cnt + 1e-8) + b_ref[...]

    return pl.pallas_call(
        body,
        grid=(B // R,),
        in_specs=[
            pl.BlockSpec((R, LP), lambda i: (i, 0)),
            pl.BlockSpec((R, 2 * LANES), lambda i: (i, 0)),
            pl.BlockSpec((2 * LANES, 8), lambda i: (0, 0)),
            pl.BlockSpec((1, 8), lambda i: (0, 0)),
        ],
        out_specs=pl.BlockSpec((R, 8), lambda i: (i, 0)),
        out_shape=jax.ShapeDtypeStruct((B, 8), jnp.float32),
    )(ids, pair_sums, sel, b8)


def kernel(input_ids, table, W, b):
    ids = input_ids.astype(jnp.int32)
    npad = LP - L
    pad_ids = (jnp.arange(B * npad, dtype=jnp.int32).reshape(B, npad)
               * 997) % (table.shape[0] - 1) + 1
    ids_p = jnp.concatenate([ids, pad_ids], axis=1).reshape(B * LP)

    w8 = jnp.pad(W.astype(jnp.float32), ((0, 8 - C), (0, 0)))
    p2 = _tc_project(table.astype(jnp.float32).T, w8)
    p1d = p2.reshape(GP * 128)

    b16 = jnp.where(jnp.arange(LANES) == 0, b[0], b[1]).astype(jnp.float32)
    return _sc_logits(ids_p, p1d, b16)[:, :C]


# speedup vs baseline: 1.0915x; 1.0019x over previous
import functools

import jax
import jax.numpy as jnp
from jax import lax
from jax.experimental import pallas as pl
from jax.experimental.pallas import tpu as pltpu
from jax.experimental.pallas import tpu_sc as plsc

B = 4096
L = 200
LP = 208
HALF = LP // 2
D = 64
C = 2
V = 1000000
NC = 2
NS = 16
NW = NC * NS
BPW = B // NW
LANES = 16
NBUF = 16
GB = 256
GP = -(-(-(-V // 128)) // GB) * GB


def _tc_project(tableT, w8):
    CB = GB * 128
    NBLK = GP * 128 // CB

    def body(tbl_ref, w_ref, out_ref):
        w2 = w_ref[pl.ds(0, 2), :].astype(jnp.bfloat16)
        blk = tbl_ref[...].astype(jnp.bfloat16)
        p = lax.dot_general(w2, blk, (((1,), (0,)), ((), ())),
                            preferred_element_type=jnp.float32)
        pb = lax.bitcast_convert_type(
            p.astype(jnp.bfloat16), jnp.uint16).astype(jnp.uint32)
        word = (pb[0:1, :] << 16) | pb[1:2, :]
        out_ref[0, :, :] = lax.bitcast_convert_type(word, jnp.float32)

    return pl.pallas_call(
        body,
        grid=(NBLK,),
        in_specs=[
            pl.BlockSpec((D, CB), lambda i: (0, i)),
            pl.BlockSpec((8, D), lambda i: (0, 0)),
        ],
        out_specs=pl.BlockSpec((1, 1, CB), lambda i: (i, 0, 0)),
        out_shape=jax.ShapeDtypeStruct((NBLK, 1, CB), jnp.float32),
    )(tableT, w8)


def _sc_logits(ids_p, p1d, b8):
    mesh = plsc.VectorSubcoreMesh(
        core_axis_name="c", subcore_axis_name="s",
        num_cores=NC, num_subcores=NS)

    @functools.partial(
        pl.kernel,
        out_type=jax.ShapeDtypeStruct((B, LANES), jnp.float32),
        mesh=mesh,
        scratch_types=(
            [pltpu.VMEM((BPW * LP,), jnp.int32)]
            + [pltpu.VMEM((LP,), jnp.float32)] * NBUF
            + [pltpu.VMEM((BPW, LANES), jnp.float32)]
            + [pltpu.VMEM((LANES,), jnp.float32)]
            + [pltpu.SemaphoreType.DMA] * NBUF
        ),
        compiler_params=pltpu.CompilerParams(
            use_tc_tiling_on_sc=False, needs_layout_passes=False),
    )
    def k(ids_hbm, p_hbm, b_hbm, out_hbm, ids_v, *rest):
        bufs = rest[:NBUF]
        sums_v = rest[NBUF]
        b_v = rest[NBUF + 1]
        sems = rest[NBUF + 2:]
        wid = lax.axis_index("s") * NC + lax.axis_index("c")
        base = wid * BPW
        pltpu.sync_copy(b_hbm, b_v)
        pltpu.sync_copy(ids_hbm.at[pl.ds(base * LP, BPW * LP)], ids_v)

        def copies(r, sbuf, sem):
            return (
                pltpu.make_async_copy(
                    p_hbm.at[ids_v.at[pl.ds(r * LP, HALF)]],
                    sbuf.at[pl.ds(0, HALF)], sem),
                pltpu.make_async_copy(
                    p_hbm.at[ids_v.at[pl.ds(r * LP + HALF, HALF)]],
                    sbuf.at[pl.ds(HALF, HALF)], sem),
            )

        def issue(r, j):
            c0, c1 = copies(r, bufs[j], sems[j])
            c0.start()
            c1.start()

        def wait(r, j):
            c0, c1 = copies(r, bufs[j], sems[j])
            c0.wait()
            c1.wait()

        tailmask = lax.iota(jnp.int32, LANES) < (L % LANES)
        lane0 = lax.iota(jnp.int32, LANES) == 0

        def accum(r, sbuf):
            z = jnp.zeros((LANES,), jnp.float32)

            def unpacked(t):
                words = sbuf[pl.ds(LANES * t, LANES)]
                pairs = plsc.bitcast(words, jnp.bfloat16)
                return plsc.unpack(pairs, format=plsc.PackFormat.INTERLEAVED)

            def nonpad(t):
                return (ids_v[pl.ds(r * LP + LANES * t, LANES)] != 0).astype(
                    jnp.float32)

            def body(t, a):
                a1, a0, ac = a
                lo, hi = unpacked(t)
                return (a1 + lo, a0 + hi, ac + nonpad(t))

            acc1, acc0, cntv = lax.fori_loop(
                0, L // LANES, body, (z, z, z), unroll=4)
            lo, hi = unpacked(L // LANES)
            acc1 = acc1 + jnp.where(tailmask, lo, 0.0)
            acc0 = acc0 + jnp.where(tailmask, hi, 0.0)
            cntv = cntv + jnp.where(tailmask, nonpad(L // LANES), 0.0)
            s0v = jnp.full((LANES,), jnp.sum(acc0))
            s1v = jnp.full((LANES,), jnp.sum(acc1))
            cnv = jnp.full((LANES,), jnp.sum(cntv))
            res = jnp.where(lane0, s0v, s1v) / (cnv + 1e-8) + b_v[...]
            sums_v[r, pl.ds(0, LANES)] = res

        for j in range(NBUF - 1):
            issue(j, j)

        def step(i, carry):
            rr = i * NBUF
            for j in range(NBUF):
                r = rr + j
                jn = (j + NBUF - 1) % NBUF

                @pl.when(r + NBUF - 1 < BPW)
                def _():
                    issue(r + NBUF - 1, jn)

                wait(r, j)
                accum(r, bufs[j])
            return carry

        lax.fori_loop(0, BPW // NBUF, step, 0)
        pltpu.sync_copy(sums_v, out_hbm.at[pl.ds(base, BPW), :])

    return k(ids_p, p1d, b8)


def kernel(input_ids, table, W, b):
    ids = input_ids.astype(jnp.int32)
    npad = LP - L
    pad_ids = (jnp.arange(B * npad, dtype=jnp.int32).reshape(B, npad)
               * 997) % (table.shape[0] - 1) + 1
    ids_p = jnp.concatenate([ids, pad_ids], axis=1).reshape(B * LP)

    w8 = jnp.pad(W.astype(jnp.float32), ((0, 8 - C), (0, 0)))
    p2 = _tc_project(table.astype(jnp.float32).T, w8)
    p1d = p2.reshape(GP * 128)

    b16 = jnp.where(jnp.arange(LANES) == 0, b[0], b[1]).astype(jnp.float32)
    return _sc_logits(ids_p, p1d, b16)[:, :C]
